# baseline (speedup 1.0000x reference)
import functools

import jax
import jax.numpy as jnp
from jax import lax
from jax.experimental import pallas as pl
from jax.experimental.pallas import tpu as pltpu
from jax.experimental.pallas import tpu_sc as plsc

_LOG2 = 0.6931471805599453

_N = 10000
_E = 160000
_F = 128
_R = 20
_H = 64
_K = 128
_NBLK = _E // _K
_NTILES = 16
_NP = 2
_AP = _N // _NP
_AROWS = _AP + 8
_ZCHUNK = 40
_NZCH = _AP // _ZCHUNK


def _ssp(x):
    return jax.nn.softplus(x) - _LOG2


_BE = 1280


def _filt_body(f_ref, rc_ref, d_ref, w1_ref, b1_ref, w2_ref, b2_ref, out_ref):
    f = f_ref[...]
    h = _ssp(jnp.dot(f, w1_ref[...], preferred_element_type=jnp.float32)
             + b1_ref[...][None, :])
    d = d_ref[0, 0, :]
    scale = rc_ref[0, 0, :] / (d * d * d * d * d)
    w = jnp.dot(h, w2_ref[...], preferred_element_type=jnp.float32)
    out_ref[...] = (w + b2_ref[...][None, :]) * scale[:, None]


def _stage1(f_ij, rcut_ij, d_ij, w1, b1, w2, b2):
    grid = (_E // _BE,)
    return pl.pallas_call(
        _filt_body,
        grid=grid,
        in_specs=[
            pl.BlockSpec((_BE, _R), lambda i: (i, 0)),
            pl.BlockSpec((1, 1, _BE), lambda i: (i, 0, 0)),
            pl.BlockSpec((1, 1, _BE), lambda i: (i, 0, 0)),
            pl.BlockSpec((_R, _F), lambda i: (0, 0)),
            pl.BlockSpec((_F,), lambda i: (0,)),
            pl.BlockSpec((_F, _F), lambda i: (0, 0)),
            pl.BlockSpec((_F,), lambda i: (0,)),
        ],
        out_specs=pl.BlockSpec((_BE, _F), lambda i: (i, 0)),
        out_shape=jax.ShapeDtypeStruct((_E, _F), jnp.float32),
    )(f_ij, rcut_ij.reshape(_E // _BE, 1, _BE), d_ij.reshape(_E // _BE, 1, _BE),
      w1, b1, w2, b2)


def _sc_body(mu2_hbm, wdiv_hbm, idxj2_hbm, idxi2_hbm, v3_hbm, d_hbm, b16_hbm,
             out_hbm, ij0, ij1, ii_v, v_v, d_v, w_v, rows0, rows1, zbuf, b_v,
             acc, sem0, sem1):
    c = lax.axis_index("c")
    s = lax.axis_index("s")
    rows = (rows0, rows1)
    sems = (sem0, sem1)
    ijs = (ij0, ij1)

    pltpu.sync_copy(b16_hbm, b_v)
    bnd = jnp.max(b_v[pl.ds(0, 16)], axis=0)

    def _zero_row(i, _):
        for r in range(12):
            zbuf[i, pl.ds(r * 16, 16)] = jnp.zeros((16,), jnp.float32)
        return _

    lax.fori_loop(0, _ZCHUNK, _zero_row, 0)

    nz = jnp.where(s < _NZCH - (_NZCH // _NTILES) * _NTILES,
                   _NZCH // _NTILES + 1, _NZCH // _NTILES)
    cols = [jnp.full((16,), j, jnp.int32) for j in range(3)]

    def _compute_block(rbuf):
        def _edge(e, _c2):
            e16 = jnp.full((16,), e * 3, jnp.int32)
            v0 = plsc.load_gather(v_v, (e16 + cols[0],))
            v1 = plsc.load_gather(v_v, (e16 + cols[1],))
            v2 = plsc.load_gather(v_v, (e16 + cols[2],))
            dd = plsc.load_gather(d_v, (jnp.full((16,), e, jnp.int32),))
            dsq = dd * dd
            t0 = 3.0 * v0
            t1 = 3.0 * v1
            t2 = 3.0 * v2
            for r in range(_H // 16):
                m0 = rbuf[e, pl.ds(r * 16, 16)]
                m1 = rbuf[e, pl.ds(_H + r * 16, 16)]
                m2 = rbuf[e, pl.ds(2 * _H + r * 16, 16)]
                w = w_v[e, pl.ds(r * 16, 16)]
                sfw = (v0 * m0 + v1 * m1 + v2 * m2) * w
                dw = dsq * w
                rbuf[e, pl.ds(r * 16, 16)] = dw * m0 - t0 * sfw
                rbuf[e, pl.ds(_H + r * 16, 16)] = dw * m1 - t1 * sfw
                rbuf[e, pl.ds(2 * _H + r * 16, 16)] = dw * m2 - t2 * sfw
            return _c2

        lax.fori_loop(0, _K, _edge, 0, unroll=8)

    def _pass(p, _0):
        def _zero_chunk(t, _):
            ch = s + _NTILES * t
            pltpu.sync_copy(zbuf, acc.at[pl.ds(ch * _ZCHUNK, _ZCHUNK)])
            return _

        lax.fori_loop(0, nz, _zero_chunk, 0)
        pltpu.sync_copy(zbuf.at[pl.ds(0, 8)], acc.at[pl.ds(_AP, 8)])
        plsc.subcore_barrier()

        lo = p * _AP
        g0 = jnp.where(p == 0, 0, bnd // _K)
        gend = jnp.where(p == 0, jnp.minimum((bnd + _K - 1) // _K, _NBLK),
                         _NBLK)
        npairs = (gend - g0 - s + 2 * _NTILES - 1) // (2 * _NTILES)

        def _base(t):
            g = jnp.clip(g0 + s + _NTILES * t, 0, _NBLK - 1)
            return g * _K, (g0 + s + _NTILES * t) < gend

        def _load_linear(t):
            base, _v = _base(t)
            pltpu.sync_copy(idxi2_hbm.at[pl.ds(p * _E + base, _K)], ii_v.at[0])
            pltpu.sync_copy(v3_hbm.at[pl.ds(base * 3, _K * 3)], v_v)
            pltpu.sync_copy(d_hbm.at[pl.ds(base, _K)], d_v)
            pltpu.sync_copy(wdiv_hbm.at[pl.ds(base, _K), pl.ds(c * _H, _H)],
                            w_v)

        def _issue_gather(t, ijbuf, rbuf, sm):
            base, _v = _base(t)
            pltpu.sync_copy(idxj2_hbm.at[pl.ds(c * _E + base, _K)],
                            ijbuf.at[0])
            pltpu.async_copy(mu2_hbm.at[ijbuf.at[0]], rbuf, sm)

        _load_linear(jnp.int32(0))
        _issue_gather(jnp.int32(0), ijs[0], rows[0], sems[0])

        def _pair(u, carry):
            for b in (0, 1):
                t = 2 * u + b
                nb = 1 - b
                _issue_gather(t + 1, ijs[nb], rows[nb], sems[nb])
                pltpu.make_async_copy(mu2_hbm.at[ijs[b].at[0]], rows[b],
                                      sems[b]).wait()
                _bs, bvalid = _base(t)

                @pl.when(bvalid)
                def _():
                    _compute_block(rows[b])
                    pltpu.sync_copy(rows[b], acc.at[ii_v.at[0]], add=True)

                _load_linear(t + 1)
            return carry

        lax.fori_loop(0, npairs, _pair, 0)
        pltpu.make_async_copy(mu2_hbm.at[ijs[0].at[0]], rows[0],
                              sems[0]).wait()

        plsc.subcore_barrier()

        def _out_chunk(t, _):
            ch = s + _NTILES * t
            pltpu.sync_copy(acc.at[pl.ds(ch * _ZCHUNK, _ZCHUNK)],
                            out_hbm.at[c, pl.ds(lo + ch * _ZCHUNK, _ZCHUNK)])
            return _

        lax.fori_loop(0, nz, _out_chunk, 0)
        plsc.subcore_barrier()
        return _0

    lax.fori_loop(0, _NP, _pass, 0)


def _stage2(mu2, wdiv, idxj2, idxi2, v3, d_ij, b16):
    mesh = plsc.VectorSubcoreMesh(core_axis_name="c", subcore_axis_name="s")
    run = pl.kernel(
        _sc_body,
        out_type=jax.ShapeDtypeStruct((2, _N, 3 * _H), jnp.float32),
        mesh=mesh,
        scratch_types=[
            pltpu.VMEM((1, _K), jnp.int32),
            pltpu.VMEM((1, _K), jnp.int32),
            pltpu.VMEM((1, _K), jnp.int32),
            pltpu.VMEM((_K * 3,), jnp.float32),
            pltpu.VMEM((_K,), jnp.float32),
            pltpu.VMEM((_K, _H), jnp.float32),
            pltpu.VMEM((_K, 3 * _H), jnp.float32),
            pltpu.VMEM((_K, 3 * _H), jnp.float32),
            pltpu.VMEM((_ZCHUNK, 3 * _H), jnp.float32),
            pltpu.VMEM((16,), jnp.int32),
            pltpu.VMEM_SHARED((_AROWS, 3 * _H), jnp.float32),
            pltpu.SemaphoreType.DMA,
            pltpu.SemaphoreType.DMA,
        ],
        compiler_params=pltpu.CompilerParams(needs_layout_passes=False,
                                             use_tc_tiling_on_sc=False),
    )
    return run(mu2, wdiv, idxj2, idxi2, v3, d_ij, b16)


_BN = 1000


def _out_body(mu_ref, t_ref, w_ref, b_ref, out_ref):
    r0 = mu_ref[0, :, 0:_H] * t_ref[0, :, 0:_H]
    r1 = mu_ref[1, :, 0:_H] * t_ref[1, :, 0:_H]
    for a in (1, 2):
        r0 = r0 + mu_ref[0, :, a*_H:(a+1)*_H] * t_ref[0, :, a*_H:(a+1)*_H]
        r1 = r1 + mu_ref[1, :, a*_H:(a+1)*_H] * t_ref[1, :, a*_H:(a+1)*_H]
    y = (jnp.dot(r0, w_ref[:_H, :], preferred_element_type=jnp.float32)
         + jnp.dot(r1, w_ref[_H:, :], preferred_element_type=jnp.float32)
         + b_ref[...][None, :])
    out_ref[...] = _ssp(y)


def _stage3(mu_r, tens, tw, tb):
    grid = (_N // _BN,)
    return pl.pallas_call(
        _out_body,
        grid=grid,
        in_specs=[
            pl.BlockSpec((2, _BN, 3 * _H), lambda i: (0, i, 0)),
            pl.BlockSpec((2, _BN, 3 * _H), lambda i: (0, i, 0)),
            pl.BlockSpec((_F, _F), lambda i: (0, 0)),
            pl.BlockSpec((_F,), lambda i: (0,)),
        ],
        out_specs=pl.BlockSpec((_BN, _F), lambda i: (i, 0)),
        out_shape=jax.ShapeDtypeStruct((_N, _F), jnp.float32),
    )(mu_r, tens, tw, tb)


@jax.jit
def kernel(q, mu_electric, f_ij, d_ij, v_ij, idx_i, idx_j, rcut_ij,
           filt_W1, filt_b1, filt_W2, filt_b2, trans_W, trans_b):
    n = mu_electric.shape[0]
    mu_r = (mu_electric.reshape(n, 3, 2, _H).transpose(2, 0, 1, 3)
            .reshape(2, n, 3 * _H))
    mu2 = mu_r.reshape(2 * n, 3 * _H)
    v3 = v_ij.reshape(-1)
    bnd = jnp.searchsorted(idx_i, _AP, side="left").astype(jnp.int32)
    b16 = jnp.full((16,), bnd, jnp.int32)
    idxj2 = jnp.concatenate([idx_j, idx_j + n])
    idxi2 = jnp.concatenate([jnp.where(idx_i < _AP, idx_i, _AP),
                             jnp.where(idx_i >= _AP, idx_i - _AP, _AP)])

    wdiv = _stage1(f_ij, rcut_ij, d_ij, filt_W1, filt_b1, filt_W2, filt_b2)
    tens = _stage2(mu2, wdiv, idxj2, idxi2, v3, d_ij, b16)
    dq = _stage3(mu_r, tens, trans_W, trans_b)
    return dq[:, None, :]


# speedup vs baseline: 1.0763x; 1.0763x over previous
import functools

import jax
import jax.numpy as jnp
from jax import lax
from jax.experimental import pallas as pl
from jax.experimental.pallas import tpu as pltpu
from jax.experimental.pallas import tpu_sc as plsc

_LOG2 = 0.6931471805599453

_N = 10000
_E = 160000
_F = 128
_R = 20
_H = 64
_K = 128
_NBLK = _E // _K
_NTILES = 16
_S = 4
_NP = 2
_AP = _N // _NP
_AROWS = _AP + 8
_ZCHUNK = 40
_NZCH = _AP // _ZCHUNK


def _ssp(x):
    return jax.nn.softplus(x) - _LOG2


_BE = 1280


def _filt_body(f_ref, rc_ref, d_ref, w1_ref, b1_ref, w2_ref, b2_ref, out_ref):
    f = f_ref[...]
    h = _ssp(jnp.dot(f, w1_ref[...], preferred_element_type=jnp.float32)
             + b1_ref[...][None, :])
    d = d_ref[0, 0, :]
    scale = rc_ref[0, 0, :] / (d * d * d * d * d)
    w = jnp.dot(h, w2_ref[...], preferred_element_type=jnp.float32)
    out_ref[...] = (w + b2_ref[...][None, :]) * scale[:, None]


def _stage1(f_ij, rcut_ij, d_ij, w1, b1, w2, b2):
    grid = (_E // _BE,)
    return pl.pallas_call(
        _filt_body,
        grid=grid,
        in_specs=[
            pl.BlockSpec((_BE, _R), lambda i: (i, 0)),
            pl.BlockSpec((1, 1, _BE), lambda i: (i, 0, 0)),
            pl.BlockSpec((1, 1, _BE), lambda i: (i, 0, 0)),
            pl.BlockSpec((_R, _F), lambda i: (0, 0)),
            pl.BlockSpec((_F,), lambda i: (0,)),
            pl.BlockSpec((_F, _F), lambda i: (0, 0)),
            pl.BlockSpec((_F,), lambda i: (0,)),
        ],
        out_specs=pl.BlockSpec((_BE, _F), lambda i: (i, 0)),
        out_shape=jax.ShapeDtypeStruct((_E, _F), jnp.float32),
    )(f_ij, rcut_ij.reshape(_E // _BE, 1, _BE), d_ij.reshape(_E // _BE, 1, _BE),
      w1, b1, w2, b2)


def _sc_body(mu2_hbm, wdiv_hbm, idxj2_hbm, idxi2_hbm, v3_hbm, d_hbm, b16_hbm,
             out_hbm, ij0, ij1, ii_v, v_v, d_v, w_v, rows0, rows1, zbuf, b_v,
             acc, sem0, sem1):
    c = lax.axis_index("c")
    s = lax.axis_index("s")
    rows = (rows0, rows1)
    sems = (sem0, sem1)
    ijs = (ij0, ij1)

    pltpu.sync_copy(b16_hbm, b_v)
    bnd = jnp.max(b_v[pl.ds(0, 16)], axis=0)

    def _zero_row(i, _):
        for r in range(12):
            zbuf[i, pl.ds(r * 16, 16)] = jnp.zeros((16,), jnp.float32)
        return _

    lax.fori_loop(0, _ZCHUNK, _zero_row, 0)

    nz = jnp.where(s < _NZCH - (_NZCH // _NTILES) * _NTILES,
                   _NZCH // _NTILES + 1, _NZCH // _NTILES)
    cols = [jnp.full((16,), j, jnp.int32) for j in range(3)]

    def _compute_block(rbuf, j):
        def _edge(e, _c2):
            e16 = jnp.full((16,), (j * _K + e) * 3, jnp.int32)
            v0 = plsc.load_gather(v_v, (e16 + cols[0],))
            v1 = plsc.load_gather(v_v, (e16 + cols[1],))
            v2 = plsc.load_gather(v_v, (e16 + cols[2],))
            dd = plsc.load_gather(d_v, (jnp.full((16,), j * _K + e,
                                                 jnp.int32),))
            dsq = dd * dd
            t0 = 3.0 * v0
            t1 = 3.0 * v1
            t2 = 3.0 * v2
            for r in range(_H // 16):
                m0 = rbuf[e, pl.ds(r * 16, 16)]
                m1 = rbuf[e, pl.ds(_H + r * 16, 16)]
                m2 = rbuf[e, pl.ds(2 * _H + r * 16, 16)]
                w = w_v[e, pl.ds(r * 16, 16)]
                sfw = (v0 * m0 + v1 * m1 + v2 * m2) * w
                dw = dsq * w
                rbuf[e, pl.ds(r * 16, 16)] = dw * m0 - t0 * sfw
                rbuf[e, pl.ds(_H + r * 16, 16)] = dw * m1 - t1 * sfw
                rbuf[e, pl.ds(2 * _H + r * 16, 16)] = dw * m2 - t2 * sfw
            return _c2

        lax.fori_loop(0, _K, _edge, 0, unroll=4)

    def _pass(p, _0):
        def _zero_chunk(t, _):
            ch = s + _NTILES * t
            pltpu.sync_copy(zbuf, acc.at[pl.ds(ch * _ZCHUNK, _ZCHUNK)])
            return _

        lax.fori_loop(0, nz, _zero_chunk, 0)
        pltpu.sync_copy(zbuf.at[pl.ds(0, 8)], acc.at[pl.ds(_AP, 8)])
        plsc.subcore_barrier()

        lo = p * _AP
        g0 = jnp.where(p == 0, 0, bnd // _K)
        gend = jnp.where(p == 0, jnp.minimum((bnd + _K - 1) // _K, _NBLK),
                         _NBLK)
        nblk = gend - g0
        rpt = (nblk + _NTILES - 1) // _NTILES
        myn = jnp.maximum(jnp.minimum(rpt, nblk - s * rpt), 0)
        nq = (rpt + _S - 1) // _S
        gmine = g0 + s * rpt

        def _stage_blk(q):
            return jnp.clip(gmine + _S * q, 0, _NBLK - _S)

        def _load_stage(q):
            gblk = _stage_blk(q)
            base = gblk * _K
            pltpu.sync_copy(idxi2_hbm.at[pl.ds(p * _NBLK + gblk, _S)], ii_v)
            pltpu.sync_copy(v3_hbm.at[pl.ds(base * 3, _S * _K * 3)], v_v)
            pltpu.sync_copy(d_hbm.at[pl.ds(base, _S * _K)], d_v)

        def _load_ij(q, ijbuf):
            gblk = _stage_blk(q)
            pltpu.sync_copy(idxj2_hbm.at[pl.ds(c * _NBLK + gblk, _S)], ijbuf)

        _load_ij(jnp.int32(0), ijs[0])
        pltpu.async_copy(mu2_hbm.at[ijs[0].at[0]], rows[0], sems[0])

        def _stagepair(u, carry):
            for qb in (0, 1):
                q = 2 * u + qb
                _load_stage(q)
                _load_ij(q + 1, ijs[1 - qb])
                base = _stage_blk(q) * _K
                for j in range(_S):
                    t = _S * q + j
                    pb = j % 2
                    npb = 1 - pb
                    if j + 1 < _S:
                        nrow = ijs[qb].at[j + 1]
                    else:
                        nrow = ijs[1 - qb].at[0]
                    pltpu.async_copy(mu2_hbm.at[nrow], rows[npb], sems[npb])
                    pltpu.make_async_copy(mu2_hbm.at[ijs[qb].at[j]], rows[pb],
                                          sems[pb]).wait()
                    pltpu.sync_copy(
                        wdiv_hbm.at[pl.ds(base + j * _K, _K),
                                    pl.ds(c * _H, _H)], w_v)

                    @pl.when(t < myn)
                    def _():
                        _compute_block(rows[pb], j)
                        pltpu.sync_copy(rows[pb], acc.at[ii_v.at[j]],
                                        add=True)

            return carry

        lax.fori_loop(0, (nq + 1) // 2, _stagepair, 0)
        pltpu.make_async_copy(mu2_hbm.at[ijs[0].at[0]], rows[0],
                              sems[0]).wait()

        plsc.subcore_barrier()

        def _out_chunk(t, _):
            ch = s + _NTILES * t
            pltpu.sync_copy(acc.at[pl.ds(ch * _ZCHUNK, _ZCHUNK)],
                            out_hbm.at[c, pl.ds(lo + ch * _ZCHUNK, _ZCHUNK)])
            return _

        lax.fori_loop(0, nz, _out_chunk, 0)
        plsc.subcore_barrier()
        return _0

    lax.fori_loop(0, _NP, _pass, 0)


def _stage2(mu2, wdiv, idxj2, idxi2, v3, d_ij, b16):
    mesh = plsc.VectorSubcoreMesh(core_axis_name="c", subcore_axis_name="s")
    run = pl.kernel(
        _sc_body,
        out_type=jax.ShapeDtypeStruct((2, _N, 3 * _H), jnp.float32),
        mesh=mesh,
        scratch_types=[
            pltpu.VMEM((_S, _K), jnp.int32),
            pltpu.VMEM((_S, _K), jnp.int32),
            pltpu.VMEM((_S, _K), jnp.int32),
            pltpu.VMEM((_S * _K * 3,), jnp.float32),
            pltpu.VMEM((_S * _K,), jnp.float32),
            pltpu.VMEM((_K, _H), jnp.float32),
            pltpu.VMEM((_K, 3 * _H), jnp.float32),
            pltpu.VMEM((_K, 3 * _H), jnp.float32),
            pltpu.VMEM((_ZCHUNK, 3 * _H), jnp.float32),
            pltpu.VMEM((16,), jnp.int32),
            pltpu.VMEM_SHARED((_AROWS, 3 * _H), jnp.float32),
            pltpu.SemaphoreType.DMA,
            pltpu.SemaphoreType.DMA,
        ],
        compiler_params=pltpu.CompilerParams(needs_layout_passes=False,
                                             use_tc_tiling_on_sc=False),
    )
    return run(mu2, wdiv, idxj2.reshape(2 * _NBLK, _K),
               idxi2.reshape(2 * _NBLK, _K), v3, d_ij, b16)


_BN = 1000


def _out_body(mu_ref, t_ref, w_ref, b_ref, out_ref):
    r0 = mu_ref[0, :, 0:_H] * t_ref[0, :, 0:_H]
    r1 = mu_ref[1, :, 0:_H] * t_ref[1, :, 0:_H]
    for a in (1, 2):
        r0 = r0 + mu_ref[0, :, a*_H:(a+1)*_H] * t_ref[0, :, a*_H:(a+1)*_H]
        r1 = r1 + mu_ref[1, :, a*_H:(a+1)*_H] * t_ref[1, :, a*_H:(a+1)*_H]
    y = (jnp.dot(r0, w_ref[:_H, :], preferred_element_type=jnp.float32)
         + jnp.dot(r1, w_ref[_H:, :], preferred_element_type=jnp.float32)
         + b_ref[...][None, :])
    out_ref[...] = _ssp(y)


def _stage3(mu_r, tens, tw, tb):
    grid = (_N // _BN,)
    return pl.pallas_call(
        _out_body,
        grid=grid,
        in_specs=[
            pl.BlockSpec((2, _BN, 3 * _H), lambda i: (0, i, 0)),
            pl.BlockSpec((2, _BN, 3 * _H), lambda i: (0, i, 0)),
            pl.BlockSpec((_F, _F), lambda i: (0, 0)),
            pl.BlockSpec((_F,), lambda i: (0,)),
        ],
        out_specs=pl.BlockSpec((_BN, _F), lambda i: (i, 0)),
        out_shape=jax.ShapeDtypeStruct((_N, _F), jnp.float32),
    )(mu_r, tens, tw, tb)


@jax.jit
def kernel(q, mu_electric, f_ij, d_ij, v_ij, idx_i, idx_j, rcut_ij,
           filt_W1, filt_b1, filt_W2, filt_b2, trans_W, trans_b):
    n = mu_electric.shape[0]
    mu_r = (mu_electric.reshape(n, 3, 2, _H).transpose(2, 0, 1, 3)
            .reshape(2, n, 3 * _H))
    mu2 = mu_r.reshape(2 * n, 3 * _H)
    v3 = v_ij.reshape(-1)
    bnd = jnp.searchsorted(idx_i, _AP, side="left").astype(jnp.int32)
    b16 = jnp.full((16,), bnd, jnp.int32)
    idxj2 = jnp.concatenate([idx_j, idx_j + n])
    idxi2 = jnp.concatenate([jnp.where(idx_i < _AP, idx_i, _AP),
                             jnp.where(idx_i >= _AP, idx_i - _AP, _AP)])

    wdiv = _stage1(f_ij, rcut_ij, d_ij, filt_W1, filt_b1, filt_W2, filt_b2)
    tens = _stage2(mu2, wdiv, idxj2, idxi2, v3, d_ij, b16)
    dq = _stage3(mu_r, tens, trans_W, trans_b)
    return dq[:, None, :]


# speedup vs baseline: 1.1210x; 1.0415x over previous
import functools

import jax
import jax.numpy as jnp
from jax import lax
from jax.experimental import pallas as pl
from jax.experimental.pallas import tpu as pltpu
from jax.experimental.pallas import tpu_sc as plsc

_LOG2 = 0.6931471805599453

_N = 10000
_E = 160000
_F = 128
_R = 20
_H = 64
_K = 128
_NBLK = _E // _K
_NTILES = 16
_S = 4
_NP = 2
_AP = _N // _NP
_AROWS = _AP + 8
_ZCHUNK = 40
_NZCH = _AP // _ZCHUNK


def _ssp(x):
    return jax.nn.softplus(x) - _LOG2


_BE = 3200


def _filt_body(f_ref, rc_ref, d_ref, w1_ref, b1_ref, w2_ref, b2_ref, out_ref):
    f = f_ref[...]
    h = _ssp(jnp.dot(f, w1_ref[...], preferred_element_type=jnp.float32)
             + b1_ref[...][None, :])
    d = d_ref[0, 0, :]
    scale = rc_ref[0, 0, :] / (d * d * d * d * d)
    w = jnp.dot(h, w2_ref[...], preferred_element_type=jnp.float32)
    out_ref[...] = (w + b2_ref[...][None, :]) * scale[:, None]


def _stage1(f_ij, rcut_ij, d_ij, w1, b1, w2, b2):
    grid = (_E // _BE,)
    return pl.pallas_call(
        _filt_body,
        grid=grid,
        in_specs=[
            pl.BlockSpec((_BE, _R), lambda i: (i, 0)),
            pl.BlockSpec((1, 1, _BE), lambda i: (i, 0, 0)),
            pl.BlockSpec((1, 1, _BE), lambda i: (i, 0, 0)),
            pl.BlockSpec((_R, _F), lambda i: (0, 0)),
            pl.BlockSpec((_F,), lambda i: (0,)),
            pl.BlockSpec((_F, _F), lambda i: (0, 0)),
            pl.BlockSpec((_F,), lambda i: (0,)),
        ],
        out_specs=pl.BlockSpec((_BE, _F), lambda i: (i, 0)),
        out_shape=jax.ShapeDtypeStruct((_E, _F), jnp.float32),
    )(f_ij, rcut_ij.reshape(_E // _BE, 1, _BE), d_ij.reshape(_E // _BE, 1, _BE),
      w1, b1, w2, b2)


def _sc_body(mu2_hbm, wdiv_hbm, idxj2_hbm, idxi2_hbm, v3_hbm, d_hbm, b16_hbm,
             out_hbm, ij0, ij1, ii_v, v_v, d_v, w_v, rows0, rows1, zbuf, b_v,
             acc, sem0, sem1):
    c = lax.axis_index("c")
    s = lax.axis_index("s")
    rows = (rows0, rows1)
    sems = (sem0, sem1)
    ijs = (ij0, ij1)

    pltpu.sync_copy(b16_hbm, b_v)
    bnd = jnp.max(b_v[pl.ds(0, 16)], axis=0)

    def _zero_row(i, _):
        for r in range(12):
            zbuf[i, pl.ds(r * 16, 16)] = jnp.zeros((16,), jnp.float32)
        return _

    lax.fori_loop(0, _ZCHUNK, _zero_row, 0)

    nz = jnp.where(s < _NZCH - (_NZCH // _NTILES) * _NTILES,
                   _NZCH // _NTILES + 1, _NZCH // _NTILES)
    cols = [jnp.full((16,), j, jnp.int32) for j in range(3)]

    def _compute_block(rbuf, j):
        def _edge(e, _c2):
            e16 = jnp.full((16,), (j * _K + e) * 3, jnp.int32)
            v0 = plsc.load_gather(v_v, (e16 + cols[0],))
            v1 = plsc.load_gather(v_v, (e16 + cols[1],))
            v2 = plsc.load_gather(v_v, (e16 + cols[2],))
            dd = plsc.load_gather(d_v, (jnp.full((16,), j * _K + e,
                                                 jnp.int32),))
            dsq = dd * dd
            t0 = 3.0 * v0
            t1 = 3.0 * v1
            t2 = 3.0 * v2
            for r in range(_H // 16):
                m0 = rbuf[e, pl.ds(r * 16, 16)]
                m1 = rbuf[e, pl.ds(_H + r * 16, 16)]
                m2 = rbuf[e, pl.ds(2 * _H + r * 16, 16)]
                w = w_v[e, pl.ds(r * 16, 16)]
                sfw = (v0 * m0 + v1 * m1 + v2 * m2) * w
                dw = dsq * w
                rbuf[e, pl.ds(r * 16, 16)] = dw * m0 - t0 * sfw
                rbuf[e, pl.ds(_H + r * 16, 16)] = dw * m1 - t1 * sfw
                rbuf[e, pl.ds(2 * _H + r * 16, 16)] = dw * m2 - t2 * sfw
            return _c2

        lax.fori_loop(0, _K, _edge, 0, unroll=4)

    def _pass(p, _0):
        def _zero_chunk(t, _):
            ch = s + _NTILES * t
            pltpu.sync_copy(zbuf, acc.at[pl.ds(ch * _ZCHUNK, _ZCHUNK)])
            return _

        lax.fori_loop(0, nz, _zero_chunk, 0)
        pltpu.sync_copy(zbuf.at[pl.ds(0, 8)], acc.at[pl.ds(_AP, 8)])
        plsc.subcore_barrier()

        lo = p * _AP
        g0 = jnp.where(p == 0, 0, bnd // _K)
        gend = jnp.where(p == 0, jnp.minimum((bnd + _K - 1) // _K, _NBLK),
                         _NBLK)
        nblk = gend - g0
        rpt = (nblk + _NTILES - 1) // _NTILES
        myn = jnp.maximum(jnp.minimum(rpt, nblk - s * rpt), 0)
        nq = (rpt + _S - 1) // _S
        gmine = g0 + s * rpt

        def _stage_blk(q):
            return jnp.clip(gmine + _S * q, 0, _NBLK - _S)

        def _load_stage(q):
            gblk = _stage_blk(q)
            base = gblk * _K
            pltpu.sync_copy(idxi2_hbm.at[pl.ds(p * _NBLK + gblk, _S)], ii_v)
            pltpu.sync_copy(v3_hbm.at[pl.ds(base * 3, _S * _K * 3)], v_v)
            pltpu.sync_copy(d_hbm.at[pl.ds(base, _S * _K)], d_v)

        def _load_ij(q, ijbuf):
            gblk = _stage_blk(q)
            pltpu.sync_copy(idxj2_hbm.at[pl.ds(c * _NBLK + gblk, _S)], ijbuf)

        _load_ij(jnp.int32(0), ijs[0])
        pltpu.async_copy(mu2_hbm.at[ijs[0].at[0]], rows[0], sems[0])

        def _stagepair(u, carry):
            for qb in (0, 1):
                q = 2 * u + qb
                _load_stage(q)
                _load_ij(q + 1, ijs[1 - qb])
                base = _stage_blk(q) * _K
                for j in range(_S):
                    t = _S * q + j
                    pb = j % 2
                    npb = 1 - pb
                    if j + 1 < _S:
                        nrow = ijs[qb].at[j + 1]
                    else:
                        nrow = ijs[1 - qb].at[0]
                    pltpu.async_copy(mu2_hbm.at[nrow], rows[npb], sems[npb])
                    pltpu.make_async_copy(mu2_hbm.at[ijs[qb].at[j]], rows[pb],
                                          sems[pb]).wait()
                    pltpu.sync_copy(
                        wdiv_hbm.at[pl.ds(base + j * _K, _K),
                                    pl.ds(c * _H, _H)], w_v)

                    @pl.when(t < myn)
                    def _():
                        _compute_block(rows[pb], j)
                        pltpu.sync_copy(rows[pb], acc.at[ii_v.at[j]],
                                        add=True)

            return carry

        lax.fori_loop(0, (nq + 1) // 2, _stagepair, 0)
        pltpu.make_async_copy(mu2_hbm.at[ijs[0].at[0]], rows[0],
                              sems[0]).wait()

        plsc.subcore_barrier()

        def _out_chunk(t, _):
            ch = s + _NTILES * t
            pltpu.sync_copy(acc.at[pl.ds(ch * _ZCHUNK, _ZCHUNK)],
                            out_hbm.at[c, pl.ds(lo + ch * _ZCHUNK, _ZCHUNK)])
            return _

        lax.fori_loop(0, nz, _out_chunk, 0)
        plsc.subcore_barrier()
        return _0

    lax.fori_loop(0, _NP, _pass, 0)


def _stage2(mu2, wdiv, idxj2, idxi2, v3, d_ij, b16):
    mesh = plsc.VectorSubcoreMesh(core_axis_name="c", subcore_axis_name="s")
    run = pl.kernel(
        _sc_body,
        out_type=jax.ShapeDtypeStruct((2, _N, 3 * _H), jnp.float32),
        mesh=mesh,
        scratch_types=[
            pltpu.VMEM((_S, _K), jnp.int32),
            pltpu.VMEM((_S, _K), jnp.int32),
            pltpu.VMEM((_S, _K), jnp.int32),
            pltpu.VMEM((_S * _K * 3,), jnp.float32),
            pltpu.VMEM((_S * _K,), jnp.float32),
            pltpu.VMEM((_K, _H), jnp.float32),
            pltpu.VMEM((_K, 3 * _H), jnp.float32),
            pltpu.VMEM((_K, 3 * _H), jnp.float32),
            pltpu.VMEM((_ZCHUNK, 3 * _H), jnp.float32),
            pltpu.VMEM((16,), jnp.int32),
            pltpu.VMEM_SHARED((_AROWS, 3 * _H), jnp.float32),
            pltpu.SemaphoreType.DMA,
            pltpu.SemaphoreType.DMA,
        ],
        compiler_params=pltpu.CompilerParams(needs_layout_passes=False,
                                             use_tc_tiling_on_sc=False),
    )
    return run(mu2, wdiv, idxj2.reshape(2 * _NBLK, _K),
               idxi2.reshape(2 * _NBLK, _K), v3, d_ij, b16)


_BN = 1000


def _out_body(mu_ref, t_ref, w_ref, b_ref, out_ref):
    r0 = mu_ref[0, :, 0:_H] * t_ref[0, :, 0:_H]
    r1 = mu_ref[1, :, 0:_H] * t_ref[1, :, 0:_H]
    for a in (1, 2):
        r0 = r0 + mu_ref[0, :, a*_H:(a+1)*_H] * t_ref[0, :, a*_H:(a+1)*_H]
        r1 = r1 + mu_ref[1, :, a*_H:(a+1)*_H] * t_ref[1, :, a*_H:(a+1)*_H]
    y = (jnp.dot(r0, w_ref[:_H, :], preferred_element_type=jnp.float32)
         + jnp.dot(r1, w_ref[_H:, :], preferred_element_type=jnp.float32)
         + b_ref[...][None, :])
    out_ref[...] = _ssp(y)


def _stage3(mu_r, tens, tw, tb):
    grid = (_N // _BN,)
    return pl.pallas_call(
        _out_body,
        grid=grid,
        in_specs=[
            pl.BlockSpec((2, _BN, 3 * _H), lambda i: (0, i, 0)),
            pl.BlockSpec((2, _BN, 3 * _H), lambda i: (0, i, 0)),
            pl.BlockSpec((_F, _F), lambda i: (0, 0)),
            pl.BlockSpec((_F,), lambda i: (0,)),
        ],
        out_specs=pl.BlockSpec((_BN, _F), lambda i: (i, 0)),
        out_shape=jax.ShapeDtypeStruct((_N, _F), jnp.float32),
    )(mu_r, tens, tw, tb)


@jax.jit
def kernel(q, mu_electric, f_ij, d_ij, v_ij, idx_i, idx_j, rcut_ij,
           filt_W1, filt_b1, filt_W2, filt_b2, trans_W, trans_b):
    n = mu_electric.shape[0]
    mu_r = (mu_electric.reshape(n, 3, 2, _H).transpose(2, 0, 1, 3)
            .reshape(2, n, 3 * _H))
    mu2 = mu_r.reshape(2 * n, 3 * _H)
    v3 = v_ij.reshape(-1)
    bnd = jnp.searchsorted(idx_i, _AP, side="left").astype(jnp.int32)
    b16 = jnp.full((16,), bnd, jnp.int32)
    idxj2 = jnp.concatenate([idx_j, idx_j + n])
    idxi2 = jnp.concatenate([jnp.where(idx_i < _AP, idx_i, _AP),
                             jnp.where(idx_i >= _AP, idx_i - _AP, _AP)])

    wdiv = _stage1(f_ij, rcut_ij, d_ij, filt_W1, filt_b1, filt_W2, filt_b2)
    tens = _stage2(mu2, wdiv, idxj2, idxi2, v3, d_ij, b16)
    dq = _stage3(mu_r, tens, trans_W, trans_b)
    return dq[:, None, :]
